# Initial kernel scaffold; baseline (speedup 1.0000x reference)
import jax
import jax.numpy as jnp
from jax.experimental import pallas as pl


def kernel(embeddings, target):
    raise NotImplementedError("write your pallas kernel here")


# speedup vs baseline: 18.5962x; 18.5962x over previous
import functools

import jax
import jax.numpy as jnp
from jax.experimental import pallas as pl

_MARGIN = 1.0
_BETA = 1.2
_EPS = 1e-6


def _loss_kernel(e_ref, trow_ref, tcol_ref, out_ref):
    e = e_ref[...]
    n_pts, k = e.shape
    g = jax.lax.dot_general(
        e, e, (((1,), (1,)), ((), ())),
        preferred_element_type=jnp.float32,
        precision=jax.lax.Precision.HIGHEST,
    )
    sq = jnp.sum(e * e, axis=1, keepdims=True)
    sm = jnp.sum(e, axis=1, keepdims=True)
    sq_t = jnp.transpose(sq)
    sm_t = jnp.transpose(sm)
    d2 = sq + sq_t - 2.0 * g + (2.0 * _EPS) * (sm - sm_t) + (k * _EPS * _EPS)
    d = jnp.sqrt(jnp.maximum(d2, 0.0))

    row = jax.lax.broadcasted_iota(jnp.int32, (n_pts, n_pts), 0)
    col = jax.lax.broadcasted_iota(jnp.int32, (n_pts, n_pts), 1)
    upper = col > row
    same = trow_ref[...] == tcol_ref[...]

    pos = jnp.maximum(d - (_BETA - _MARGIN), 0.0)
    neg = jnp.maximum((_BETA + _MARGIN) - d, 0.0)
    contrib = jnp.where(upper, jnp.where(same, pos, neg), 0.0)
    out_ref[...] = jnp.sum(contrib).reshape(1, 1)


@functools.partial(jax.jit, static_argnames=())
def kernel(embeddings, target):
    n = embeddings.shape[0]
    t = target.astype(jnp.int32)
    trow = t.reshape(n, 1)
    tcol = t.reshape(1, n)
    out = pl.pallas_call(
        _loss_kernel,
        out_shape=jax.ShapeDtypeStruct((1, 1), jnp.float32),
    )(embeddings, trow, tcol)
    return out[0, 0]
